# baseline (speedup 1.0000x reference)
import jax
import jax.numpy as jnp
from jax.experimental import pallas as pl

_SEQ_BLK = 256


def _pe_add_kernel(x_ref, pe_ref, o_ref):
    o_ref[...] = x_ref[...] + pe_ref[...][None, :, :] * 0.002


def kernel(x, pe):
    bz, lens, d = x.shape
    pe2 = pe[:lens, 0, :]
    grid = (lens // _SEQ_BLK,)
    return pl.pallas_call(
        _pe_add_kernel,
        grid=grid,
        in_specs=[
            pl.BlockSpec((bz, _SEQ_BLK, d), lambda s: (0, s, 0)),
            pl.BlockSpec((_SEQ_BLK, d), lambda s: (s, 0)),
        ],
        out_specs=pl.BlockSpec((bz, _SEQ_BLK, d), lambda s: (0, s, 0)),
        out_shape=jax.ShapeDtypeStruct((bz, lens, d), x.dtype),
    )(x, pe2)
